# baseline (speedup 1.0000x reference)
import functools

import jax
import jax.numpy as jnp
import numpy as np
from jax import lax
from jax.experimental import pallas as pl
from jax.experimental.pallas import tpu as pltpu
from jax.experimental.pallas import tpu_sc as plsc

_K = 8
_BR = 256

_NC = 2
_NS = 16
_NW = _NC * _NS
_L = 16
_R = 16

_SC_ROWS = 4096

_LOG1P = (
    6.057847667939598e-09, 0.9999987830867273, -0.49995894468480306,
    0.3327853380006574, -0.24618967719166315, 0.18421386356488162,
    -0.12447194563436599, 0.06573552558543269, -0.0226280072114605,
    0.003662242215796141,
)
_LN2 = 0.6931471805599453


def _vlog(y):
    bits = lax.bitcast_convert_type(y, jnp.int32)
    e = (bits >> 23) - 127
    f = lax.bitcast_convert_type((bits & 0x007FFFFF) | 0x3F800000, jnp.float32)
    t = f - 1.0
    p = _LOG1P[-1] * t + _LOG1P[-2]
    for c in _LOG1P[-3::-1]:
        p = p * t + c
    return e.astype(jnp.float32) * _LN2 + p


def _mix_logprob(xv, mk, nk, ck):
    ls = [(xv - mk[k]) * (xv - mk[k]) * nk[k] + ck[k] for k in range(_K)]
    m0 = jnp.maximum(jnp.maximum(ls[0], ls[1]), jnp.maximum(ls[2], ls[3]))
    m1 = jnp.maximum(jnp.maximum(ls[4], ls[5]), jnp.maximum(ls[6], ls[7]))
    lmax = jnp.maximum(m0, m1)
    s = None
    for k in range(_K):
        e = jnp.exp(ls[k] - lmax)
        s = e if s is None else s + e
    return lmax, s


def _sc_body(coef_hbm, x_hbm, o_hbm, coef_v, buf):
    wid = lax.axis_index("s") * _NC + lax.axis_index("c")
    rows_per = x_hbm.shape[0] // _NW
    z = x_hbm.shape[1]
    pltpu.sync_copy(coef_hbm, coef_v)
    base = wid * rows_per

    def chunk_body(ci, carry):
        r0 = base + ci * _R
        pltpu.sync_copy(x_hbm.at[pl.ds(r0, _R)], buf)

        def z_body(zb, c2):
            zsl = pl.ds(zb * _L, _L)
            mk = [coef_v[k, zsl] for k in range(_K)]
            nk = [coef_v[_K + k, zsl] for k in range(_K)]
            ck = [coef_v[2 * _K + k, zsl] for k in range(_K)]
            for r in range(_R):
                lmax, s = _mix_logprob(buf[r, zsl], mk, nk, ck)
                buf[r, zsl] = lmax + _vlog(s)
            return c2

        lax.fori_loop(0, z // _L, z_body, 0)
        pltpu.sync_copy(buf, o_hbm.at[pl.ds(r0, _R)])
        return carry

    lax.fori_loop(0, rows_per // _R, chunk_body, 0)


def _sc_call(coef, x):
    b, z = x.shape
    mesh = plsc.VectorSubcoreMesh(core_axis_name="c", subcore_axis_name="s")
    return pl.kernel(
        _sc_body,
        mesh=mesh,
        out_type=jax.ShapeDtypeStruct((b, z), jnp.float32),
        scratch_types=[
            pltpu.VMEM((3 * _K, z), jnp.float32),
            pltpu.VMEM((_R, z), jnp.float32),
        ],
    )(coef, x)


def _tc_body(coef_ref, x_ref, o_ref):
    x = x_ref[...]
    mk = [coef_ref[k] for k in range(_K)]
    nk = [coef_ref[_K + k] for k in range(_K)]
    ck = [coef_ref[2 * _K + k] for k in range(_K)]
    lmax, s = _mix_logprob(x, mk, nk, ck)
    o_ref[...] = lmax + jnp.log(s)


def _tc_call(coef, x):
    b, z = x.shape
    return pl.pallas_call(
        _tc_body,
        grid=(b // _BR,),
        in_specs=[
            pl.BlockSpec((3 * _K, z), lambda i: (0, 0)),
            pl.BlockSpec((_BR, z), lambda i: (i, 0)),
        ],
        out_specs=pl.BlockSpec((_BR, z), lambda i: (i, 0)),
        out_shape=jax.ShapeDtypeStruct((b, z), x.dtype),
    )(coef, x)


@jax.jit
def kernel(x, mean_list, scale_list, weight_logits):
    B, Z = x.shape
    scale = jax.nn.softplus(scale_list)
    ninv = -0.5 / (scale * scale)
    log_w = jax.nn.log_softmax(weight_logits, axis=-1)
    cns = -jnp.log(scale) - 0.5 * np.log(2.0 * np.pi) + log_w[:, None, :]
    coef = jnp.concatenate(
        [mean_list[0].T, ninv[0].T, cns[0].T], axis=0)

    if _SC_ROWS == 0:
        return _tc_call(coef, x)
    if _SC_ROWS == B:
        return _sc_call(coef, x)
    o_tc = _tc_call(coef, x[: B - _SC_ROWS])
    o_sc = _sc_call(coef, x[B - _SC_ROWS:])
    return jnp.concatenate([o_tc, o_sc], axis=0)


# speedup vs baseline: 3.1179x; 3.1179x over previous
import functools

import jax
import jax.numpy as jnp
import numpy as np
from jax import lax
from jax.experimental import pallas as pl
from jax.experimental.pallas import tpu as pltpu
from jax.experimental.pallas import tpu_sc as plsc

_K = 8
_BR = 256

_NC = 2
_NS = 16
_NW = _NC * _NS
_L = 16
_R = 16

_SC_ROWS = 512

_LOG1P = (
    6.057847667939598e-09, 0.9999987830867273, -0.49995894468480306,
    0.3327853380006574, -0.24618967719166315, 0.18421386356488162,
    -0.12447194563436599, 0.06573552558543269, -0.0226280072114605,
    0.003662242215796141,
)
_LN2 = 0.6931471805599453


def _vlog(y):
    bits = lax.bitcast_convert_type(y, jnp.int32)
    e = (bits >> 23) - 127
    f = lax.bitcast_convert_type((bits & 0x007FFFFF) | 0x3F800000, jnp.float32)
    t = f - 1.0
    p = _LOG1P[-1] * t + _LOG1P[-2]
    for c in _LOG1P[-3::-1]:
        p = p * t + c
    return e.astype(jnp.float32) * _LN2 + p


def _mix_logprob(xv, mk, nk, ck):
    ls = [(xv - mk[k]) * (xv - mk[k]) * nk[k] + ck[k] for k in range(_K)]
    m0 = jnp.maximum(jnp.maximum(ls[0], ls[1]), jnp.maximum(ls[2], ls[3]))
    m1 = jnp.maximum(jnp.maximum(ls[4], ls[5]), jnp.maximum(ls[6], ls[7]))
    lmax = jnp.maximum(m0, m1)
    s = None
    for k in range(_K):
        e = jnp.exp(ls[k] - lmax)
        s = e if s is None else s + e
    return lmax, s


def _sc_body(coef_hbm, x_hbm, o_hbm, coef_v, buf):
    wid = lax.axis_index("s") * _NC + lax.axis_index("c")
    rows_per = x_hbm.shape[0] // _NW
    z = x_hbm.shape[1]
    pltpu.sync_copy(coef_hbm, coef_v)
    base = wid * rows_per

    def chunk_body(ci, carry):
        r0 = base + ci * _R
        pltpu.sync_copy(x_hbm.at[pl.ds(r0, _R)], buf)

        def z_body(zb, c2):
            zsl = pl.ds(zb * _L, _L)
            mk = [coef_v[k, zsl] for k in range(_K)]
            nk = [coef_v[_K + k, zsl] for k in range(_K)]
            ck = [coef_v[2 * _K + k, zsl] for k in range(_K)]
            for r in range(_R):
                lmax, s = _mix_logprob(buf[r, zsl], mk, nk, ck)
                buf[r, zsl] = lmax + _vlog(s)
            return c2

        lax.fori_loop(0, z // _L, z_body, 0)
        pltpu.sync_copy(buf, o_hbm.at[pl.ds(r0, _R)])
        return carry

    lax.fori_loop(0, rows_per // _R, chunk_body, 0)


def _sc_call(coef, x):
    b, z = x.shape
    mesh = plsc.VectorSubcoreMesh(core_axis_name="c", subcore_axis_name="s")
    return pl.kernel(
        _sc_body,
        mesh=mesh,
        out_type=jax.ShapeDtypeStruct((b, z), jnp.float32),
        scratch_types=[
            pltpu.VMEM((3 * _K, z), jnp.float32),
            pltpu.VMEM((_R, z), jnp.float32),
        ],
    )(coef, x)


def _tc_body(coef_ref, x_ref, o_ref):
    x = x_ref[...]
    mk = [coef_ref[k] for k in range(_K)]
    nk = [coef_ref[_K + k] for k in range(_K)]
    ck = [coef_ref[2 * _K + k] for k in range(_K)]
    lmax, s = _mix_logprob(x, mk, nk, ck)
    o_ref[...] = lmax + jnp.log(s)


def _tc_call(coef, x):
    b, z = x.shape
    return pl.pallas_call(
        _tc_body,
        grid=(b // _BR,),
        in_specs=[
            pl.BlockSpec((3 * _K, z), lambda i: (0, 0)),
            pl.BlockSpec((_BR, z), lambda i: (i, 0)),
        ],
        out_specs=pl.BlockSpec((_BR, z), lambda i: (i, 0)),
        out_shape=jax.ShapeDtypeStruct((b, z), x.dtype),
    )(coef, x)


@jax.jit
def kernel(x, mean_list, scale_list, weight_logits):
    B, Z = x.shape
    scale = jax.nn.softplus(scale_list)
    ninv = -0.5 / (scale * scale)
    log_w = jax.nn.log_softmax(weight_logits, axis=-1)
    cns = -jnp.log(scale) - 0.5 * np.log(2.0 * np.pi) + log_w[:, None, :]
    coef = jnp.concatenate(
        [mean_list[0].T, ninv[0].T, cns[0].T], axis=0)

    if _SC_ROWS == 0:
        return _tc_call(coef, x)
    if _SC_ROWS == B:
        return _sc_call(coef, x)
    o_tc = _tc_call(coef, x[: B - _SC_ROWS])
    o_sc = _sc_call(coef, x[B - _SC_ROWS:])
    return jnp.concatenate([o_tc, o_sc], axis=0)
